# baseline (speedup 1.0000x reference)
import functools

import jax
import jax.numpy as jnp
from jax import lax
from jax.experimental import pallas as pl
from jax.experimental.pallas import tpu as pltpu
from jax.experimental.pallas import tpu_sc as plsc

PAD = 0

NC = 2
NS = 16
NW = NC * NS

GRB = 2


@functools.lru_cache(maxsize=None)
def _make_gather(b: int, l: int, vocab: int, d: int):
    per_w_b = b // NW
    n_group = per_w_b // GRB
    n_ch = (l + 15) // 16
    lpad = n_ch * 16
    tail = l - (n_ch - 1) * 16
    mesh = plsc.VectorSubcoreMesh(core_axis_name="c", subcore_axis_name="s")

    @functools.partial(
        pl.kernel,
        mesh=mesh,
        compiler_params=pltpu.CompilerParams(use_tc_tiling_on_sc=False),
        out_type=jax.ShapeDtypeStruct((b, l, d), jnp.float32),
        scratch_types=[
            pltpu.VMEM((per_w_b, l), jnp.int32),
            pltpu.VMEM((2, GRB * lpad, d), jnp.float32),
            pltpu.SemaphoreType.DMA,
            pltpu.SemaphoreType.DMA,
            pltpu.SemaphoreType.DMA,
        ],
    )
    def gather_kernel(tok_hbm, table_hbm, out_hbm, idx_v, rows_v, g0sem, g1sem, osem):
        wid = lax.axis_index("s") * NC + lax.axis_index("c")
        base_b = wid * per_w_b
        gsems = (g0sem, g1sem)
        pltpu.sync_copy(tok_hbm.at[pl.ds(base_b, per_w_b)], idx_v)

        def fire_gathers(g, buf):
            def fire(u, c):
                rb = u // n_ch
                ch = u % n_ch
                off = jnp.minimum(ch * 16, l - 16)
                idx_vec = idx_v[g * GRB + rb, pl.ds(off, 16)]
                pltpu.async_copy(
                    table_hbm.at[idx_vec],
                    rows_v.at[buf, pl.ds(rb * lpad + ch * 16, 16)],
                    gsems[buf],
                )
                return c

            lax.fori_loop(0, GRB * n_ch, fire, 0)

        def wait_gathers(buf):
            pltpu.make_async_copy(
                table_hbm.at[pl.ds(0, GRB * lpad)], rows_v.at[buf], gsems[buf]
            ).wait()

        def out_pairs(g, buf):
            pairs = []
            head = (n_ch - 1) * 16
            for rb in range(GRB):
                row = base_b + g * GRB + rb
                pairs.append((
                    rows_v.at[buf, pl.ds(rb * lpad, head)],
                    out_hbm.at[row, pl.ds(0, head)],
                ))
                pairs.append((
                    rows_v.at[buf, pl.ds(rb * lpad + head + (16 - tail), tail)],
                    out_hbm.at[row, pl.ds(head, tail)],
                ))
            return pairs

        def fire_out(g, buf):
            for src, dst in out_pairs(g, buf):
                pltpu.async_copy(src, dst, osem)

        def wait_out(g, buf):
            for src, dst in out_pairs(g, buf):
                pltpu.make_async_copy(src, dst, osem).wait()

        fire_gathers(0, 0)
        fire_gathers(1, 1)

        def step(t, carry, last):
            for buf in (0, 1):
                g = 2 * t + buf
                wait_gathers(buf)
                fire_out(g, buf)
                wait_out(g, buf)
                if not last:
                    fire_gathers(g + 2, buf)
            return carry

        lax.fori_loop(0, n_group // 2 - 1, lambda t, c: step(t, c, False), 0)
        step(n_group // 2 - 1, 0, True)

    return gather_kernel


def kernel(tokens, table):
    b, l = tokens.shape
    vocab, d = table.shape
    features = _make_gather(b, l, vocab, d)(tokens, table)
    padding_masks = (tokens == PAD)[:, None, None, :]
    sequential_masks = jnp.triu(jnp.ones((l, l), dtype=bool), k=1)
    return features, padding_masks, sequential_masks


# speedup vs baseline: 1.3250x; 1.3250x over previous
import functools

import jax
import jax.numpy as jnp
from jax import lax
from jax.experimental import pallas as pl
from jax.experimental.pallas import tpu as pltpu
from jax.experimental.pallas import tpu_sc as plsc

PAD = 0

NC = 2
NS = 16
NW = NC * NS

GRB = 2


@functools.lru_cache(maxsize=None)
def _make_gather(b: int, l: int, vocab: int, d: int):
    per_w_b = b // NW
    n_group = per_w_b // GRB
    n_ch = (l + 15) // 16
    lpad = n_ch * 16
    tail = l - (n_ch - 1) * 16
    mesh = plsc.VectorSubcoreMesh(core_axis_name="c", subcore_axis_name="s")

    @functools.partial(
        pl.kernel,
        mesh=mesh,
        compiler_params=pltpu.CompilerParams(use_tc_tiling_on_sc=False),
        out_type=jax.ShapeDtypeStruct((b * l, 2 * d), jnp.float32),
        scratch_types=[
            pltpu.VMEM((per_w_b, l), jnp.int32),
            pltpu.VMEM((2, GRB * lpad, d), jnp.float32),
            pltpu.SemaphoreType.DMA,
            pltpu.SemaphoreType.DMA,
            pltpu.SemaphoreType.DMA,
        ],
    )
    def gather_kernel(tok_hbm, table_hbm, out_hbm, idx_v, rows_v, g0sem, g1sem, osem):
        wid = lax.axis_index("s") * NC + lax.axis_index("c")
        base_b = wid * per_w_b
        gsems = (g0sem, g1sem)
        pltpu.sync_copy(tok_hbm.at[pl.ds(base_b, per_w_b)], idx_v)

        def fire_gathers(g, buf):
            def fire(u, c):
                rb = u // n_ch
                ch = u % n_ch
                off = jnp.minimum(ch * 16, l - 16)
                idx_vec = idx_v[g * GRB + rb, pl.ds(off, 16)]
                pltpu.async_copy(
                    table_hbm.at[idx_vec],
                    rows_v.at[buf, pl.ds(rb * lpad + ch * 16, 16)],
                    gsems[buf],
                )
                return c

            lax.fori_loop(0, GRB * n_ch, fire, 0)

        def wait_gathers(buf):
            pltpu.make_async_copy(
                table_hbm.at[pl.ds(0, GRB * lpad)], rows_v.at[buf], gsems[buf]
            ).wait()

        def out_pairs(g, buf):
            pairs = []
            head = (n_ch - 1) * 16
            for rb in range(GRB):
                tok0 = (base_b + g * GRB + rb) * l
                pairs.append((
                    rows_v.at[buf, pl.ds(rb * lpad, head)],
                    out_hbm.at[pl.ds(tok0, head), pl.ds(0, d)],
                ))
                pairs.append((
                    rows_v.at[buf, pl.ds(rb * lpad + head + (16 - tail), tail)],
                    out_hbm.at[pl.ds(tok0 + head, tail), pl.ds(0, d)],
                ))
            return pairs

        def fire_out(g, buf):
            for src, dst in out_pairs(g, buf):
                pltpu.async_copy(src, dst, osem)

        def wait_out(g, buf):
            for src, dst in out_pairs(g, buf):
                pltpu.make_async_copy(src, dst, osem).wait()

        fire_gathers(0, 0)
        fire_gathers(1, 1)

        def step(t, carry, last):
            for buf in (0, 1):
                g = 2 * t + buf
                wait_gathers(buf)
                fire_out(g, buf)
                wait_out(g, buf)
                if not last:
                    fire_gathers(g + 2, buf)
            return carry

        lax.fori_loop(0, n_group // 2 - 1, lambda t, c: step(t, c, False), 0)
        step(n_group // 2 - 1, 0, True)

    return gather_kernel


def kernel(tokens, table):
    b, l = tokens.shape
    vocab, d = table.shape
    padded = _make_gather(b, l, vocab, d)(tokens, table)
    features = padded[:, :d].reshape(b, l, d)
    padding_masks = (tokens == PAD)[:, None, None, :]
    sequential_masks = jnp.triu(jnp.ones((l, l), dtype=bool), k=1)
    return features, padding_masks, sequential_masks
